# baseline (speedup 1.0000x reference)
import jax
import jax.numpy as jnp
from jax import lax
from jax.experimental import pallas as pl
from jax.experimental.pallas import tpu as pltpu
from jax.experimental.pallas import tpu_sc as plsc

_CHUNK = 128
_L = 16


def _glove_sc_t(B, D, NC, NS):
    NW = NC * NS
    bpw = B // NW
    n_chunks = bpw // _CHUNK
    mesh = plsc.VectorSubcoreMesh(
        core_axis_name="c", subcore_axis_name="s",
        num_cores=NC, num_subcores=NS)

    def body(left_hbm, right_hbm, leT, lbT, reT, rbT, out_hbm,
             idx_l, idx_r, lv, rv, bl, br, outv, sem_b, sem_d):
        wid = lax.axis_index("s") * NC + lax.axis_index("c")
        cbase = wid * n_chunks
        pltpu.sync_copy(left_hbm.at[pl.ds(cbase, n_chunks)], idx_l)
        pltpu.sync_copy(right_hbm.at[pl.ds(cbase, n_chunks)], idx_r)

        bias_hs = []
        for j in range(n_chunks):
            sl = pl.ds(j * _CHUNK, _CHUNK)
            bias_hs.append(pltpu.async_copy(lbT.at[0].at[idx_l.at[j]], bl.at[sl], sem_b))
            bias_hs.append(pltpu.async_copy(rbT.at[0].at[idx_r.at[j]], br.at[sl], sem_b))

        def dloop(dd, _):
            hs = []
            for j in range(n_chunks):
                sl = pl.ds(j * _CHUNK, _CHUNK)
                hs.append(pltpu.async_copy(leT.at[dd].at[idx_l.at[j]], lv.at[dd, sl], sem_d))
                hs.append(pltpu.async_copy(reT.at[dd].at[idx_r.at[j]], rv.at[dd, sl], sem_d))
            for h in hs:
                h.wait()
            return 0

        lax.fori_loop(0, D, dloop, 0)
        for h in bias_hs:
            h.wait()

        def gbody(k, _):
            sl = pl.ds(k * _L, _L)

            def dsum(dd, acc):
                return acc + lv[dd, sl] * rv[dd, sl]

            acc = lax.fori_loop(0, D, dsum, jnp.zeros((_L,), jnp.float32))
            outv[sl] = acc + bl[sl] + br[sl]
            return 0

        lax.fori_loop(0, bpw // _L, gbody, 0)
        pltpu.sync_copy(outv, out_hbm.at[pl.ds(wid * bpw, bpw)])

    return pl.kernel(
        body,
        out_type=jax.ShapeDtypeStruct((B,), jnp.float32),
        mesh=mesh,
        compiler_params=pltpu.CompilerParams(
            needs_layout_passes=False, use_tc_tiling_on_sc=False),
        scratch_types=[
            pltpu.VMEM((n_chunks, _CHUNK), jnp.int32),
            pltpu.VMEM((n_chunks, _CHUNK), jnp.int32),
            pltpu.VMEM((D, bpw), jnp.float32),
            pltpu.VMEM((D, bpw), jnp.float32),
            pltpu.VMEM((bpw,), jnp.float32),
            pltpu.VMEM((bpw,), jnp.float32),
            pltpu.VMEM((bpw,), jnp.float32),
            pltpu.SemaphoreType.DMA,
            pltpu.SemaphoreType.DMA,
        ],
    )


def kernel(left, right, l_emb, l_bias, r_emb, r_bias):
    (B,) = left.shape
    V, D = l_emb.shape
    info = plsc.get_sparse_core_info()
    NC, NS = info.num_cores, info.num_subcores
    left2d = left.astype(jnp.int32).reshape(B // _CHUNK, _CHUNK)
    right2d = right.astype(jnp.int32).reshape(B // _CHUNK, _CHUNK)
    fn = _glove_sc_t(B, D, NC, NS)
    return fn(left2d, right2d, l_emb.T, l_bias.T, r_emb.T, r_bias.T)
